# baseline (speedup 1.0000x reference)
import functools

import numpy as np

import jax
import jax.numpy as jnp
from jax import lax
from jax.experimental import pallas as pl
from jax.experimental.pallas import tpu as pltpu
from jax.experimental.pallas import tpu_sc as plsc

_N = 50000
_K = 25000
_D = 128
_BS = 4
_NB = 2048
_NBLK = 25

_ROWS = _BS * _K
_CHUNK = 184
_NCHUNK = 544
_LAST_BASE = _ROWS - _CHUNK


def _score_body(h_ref, w_ref, b_ref, lg_ref):
    n = pl.program_id(1)
    hblk = h_ref[0]
    lg = jnp.dot(hblk.astype(jnp.bfloat16), w_ref[...].astype(jnp.bfloat16),
                 preferred_element_type=jnp.float32)[:, 0] + b_ref[0]
    s = jax.nn.sigmoid(lg)
    node = n * _NB + lax.broadcasted_iota(jnp.int32, (_NB,), 0)
    lg_ref[0] = jnp.where(node < _N, s, -1.0).reshape(_NB // _D, _D)


def _scores_l1(h, W, b):
    return pl.pallas_call(
        _score_body,
        grid=(_BS, _NBLK),
        in_specs=[
            pl.BlockSpec((1, _NB, _D), lambda bb, n: (bb, n, 0)),
            pl.BlockSpec((_D, 1), lambda bb, n: (0, 0)),
            pl.BlockSpec(memory_space=pltpu.SMEM),
        ],
        out_specs=pl.BlockSpec((1, _NB // _D, _D), lambda bb, n: (bb, n, 0)),
        out_shape=jax.ShapeDtypeStruct((_BS, 400, _D), jnp.float32),
    )(h, W, b)


def _sort_body(lg_ref, idx_ref, ssc_ref, kv_key, kv_val):
    b = pl.program_id(0)
    rl = lax.broadcasted_iota(jnp.int32, (64, 128), 0)
    cl = lax.broadcasted_iota(jnp.int32, (64, 128), 1)
    gl = rl * 128 + cl

    def ce(key, val, pk, pv, m):
        sf = (key > pk) | ((key == pk) & (val < pv))
        keep = sf == m
        return jnp.where(keep, key, pk), jnp.where(keep, val, pv)

    def stage_in(key, val, j, dirm):
        lower = (gl & j) == 0
        if j < 128:
            ax, s_lo, s_hi = 1, 128 - j, j
        else:
            d = j // 128
            ax, s_lo, s_hi = 0, 64 - d, d
        pk = jnp.where(lower, pltpu.roll(key, s_lo, ax),
                       pltpu.roll(key, s_hi, ax))
        pv = jnp.where(lower, pltpu.roll(val, s_lo, ax),
                       pltpu.roll(val, s_hi, ax))
        return ce(key, val, pk, pv, lower == dirm)

    def cross(ka, va, kb, vb, dirb):
        sf = (ka > kb) | ((ka == kb) & (va < vb))
        keep = sf == dirb
        nka, nva = jnp.where(keep, ka, kb), jnp.where(keep, va, vb)
        nkb, nvb = jnp.where(keep, kb, ka), jnp.where(keep, vb, va)
        return nka, nva, nkb, nvb

    in_sched = []
    for p in range(1, 13):
        k = 1 << p
        j = k >> 1
        while j >= 1:
            in_sched.append((j, k))
            j >>= 1
    tail = [4096 >> t for t in range(13)]

    def pass_a(cc, carry):
        c0 = 2 * cc
        k0 = lg_ref[0, pl.ds(c0 * 64, 64), :]
        v0 = gl + c0 * 8192 + b * _N
        k1 = lg_ref[0, pl.ds(c0 * 64 + 64, 64), :]
        v1 = v0 + 8192
        for (j, k) in in_sched:
            dm = (gl & k) == 0
            k0, v0 = stage_in(k0, v0, j, dm)
            k1, v1 = stage_in(k1, v1, j, dm)
        for j in tail:
            k0, v0 = stage_in(k0, v0, j, True)
            k1, v1 = stage_in(k1, v1, j, False)
        kv_key[pl.ds(c0 * 64, 64), :] = k0
        kv_val[pl.ds(c0 * 64, 64), :] = v0
        kv_key[pl.ds(c0 * 64 + 64, 64), :] = k1
        kv_val[pl.ds(c0 * 64 + 64, 64), :] = v1
        return carry

    def make_cross_pass(c0_of, dist, kshift, with_tail, to_out):
        def body(cp, carry):
            c0 = c0_of(cp)
            oa, ob = c0 * 64, (c0 + dist) * 64
            ka = kv_key[pl.ds(oa, 64), :]
            va = kv_val[pl.ds(oa, 64), :]
            kb = kv_key[pl.ds(ob, 64), :]
            vb = kv_val[pl.ds(ob, 64), :]
            dirb = True if kshift is None else ((c0 >> kshift) & 1) == 0
            ka, va, kb, vb = cross(ka, va, kb, vb, dirb)
            if with_tail:
                for j in tail:
                    ka, va = stage_in(ka, va, j, dirb)
                    kb, vb = stage_in(kb, vb, j, dirb)
            if to_out:
                ssc_ref[0, pl.ds(oa, 64), :] = ka
                idx_ref[0, pl.ds(oa, 64), :] = va
                ssc_ref[0, pl.ds(ob, 64), :] = kb
                idx_ref[0, pl.ds(ob, 64), :] = vb
            else:
                kv_key[pl.ds(oa, 64), :] = ka
                kv_val[pl.ds(oa, 64), :] = va
                kv_key[pl.ds(ob, 64), :] = kb
                kv_val[pl.ds(ob, 64), :] = vb
            return carry

        return body

    d1 = lambda cp: 2 * cp
    d2 = lambda cp: (cp & 1) + (cp >> 1) * 4
    d4 = lambda cp: cp

    lax.fori_loop(0, 4, pass_a, 0)
    lax.fori_loop(0, 4, make_cross_pass(d1, 1, 1, True, False), 0)
    lax.fori_loop(0, 4, make_cross_pass(d2, 2, 2, False, False), 0)
    lax.fori_loop(0, 4, make_cross_pass(d1, 1, 2, True, False), 0)
    lax.fori_loop(0, 4, make_cross_pass(d4, 4, None, False, False), 0)
    lax.fori_loop(0, 4, make_cross_pass(d2, 2, None, False, False), 0)
    lax.fori_loop(0, 4, make_cross_pass(d1, 1, None, True, True), 0)


def _topk_sort(scores_l1):
    f = pl.pallas_call(
        _sort_body,
        grid=(_BS,),
        in_specs=[pl.BlockSpec((1, 512, 128), lambda b: (b, 0, 0))],
        out_specs=[
            pl.BlockSpec((1, 512, 128), lambda b: (b, 0, 0)),
            pl.BlockSpec((1, 512, 128), lambda b: (b, 0, 0)),
        ],
        out_shape=[
            jax.ShapeDtypeStruct((_BS, 512, 128), jnp.int32),
            jax.ShapeDtypeStruct((_BS, 512, 128), jnp.float32),
        ],
        scratch_shapes=[
            pltpu.VMEM((512, 128), jnp.float32),
            pltpu.VMEM((512, 128), jnp.int32),
        ],
    )
    return f(scores_l1)


def _gather_body(h_hbm, gidx_hbm, ssc_hbm, out_hbm, idx_v, sc_v, rows_v, sem):
    wid = lax.axis_index("s") * 2 + lax.axis_index("c")

    def scale_rows(r8, carry):
        for u in range(8):
            r = r8 * 8 + u
            s16 = sc_v[r]
            for q in range(8):
                rows_v[r, pl.ds(q * 16, 16)] = (
                    rows_v[r, pl.ds(q * 16, 16)] * s16)
        return carry

    def chunk(cc, carry):
        c = wid * 17 + cc
        base = jnp.where(c == _NCHUNK - 1, _LAST_BASE, c * _CHUNK)
        pltpu.sync_copy(gidx_hbm.at[pl.ds(base, _CHUNK)], idx_v)
        pltpu.sync_copy(ssc_hbm.at[pl.ds(base, _CHUNK)], sc_v)
        pltpu.async_copy(h_hbm.at[idx_v], rows_v, sem).wait()
        lax.fori_loop(0, _CHUNK // 8, scale_rows, 0)
        pltpu.sync_copy(rows_v, out_hbm.at[pl.ds(base, _CHUNK)])
        return carry

    lax.fori_loop(0, 17, chunk, 0)


def _sc_gather(h_flat, gidx, ssc):
    gk = functools.partial(
        pl.kernel,
        mesh=plsc.VectorSubcoreMesh(core_axis_name="c", subcore_axis_name="s"),
        out_type=jax.ShapeDtypeStruct((_ROWS, _D), jnp.float32),
        scratch_types=[
            pltpu.VMEM((_CHUNK,), jnp.int32),
            pltpu.VMEM((_CHUNK, 16), jnp.float32),
            pltpu.VMEM((_CHUNK, _D), jnp.float32),
            pltpu.SemaphoreType.DMA,
        ],
    )(_gather_body)
    return gk(h_flat, gidx, ssc)


def kernel(h, W, b):
    s400 = _scores_l1(h, W, b)
    scores_l1 = jnp.concatenate(
        [s400, jnp.full((_BS, 112, _D), -1.0, jnp.float32)], axis=1)
    idx_l1, ssc_l1 = _topk_sort(scores_l1)
    gidx = idx_l1.reshape(_BS, 512 * _D)[:, :_K].reshape(-1)
    ssc = ssc_l1.reshape(_BS, 512 * _D)[:, :_K].reshape(-1)
    ssc_rep = jnp.broadcast_to(ssc[:, None], (_ROWS, 16))
    out_flat = _sc_gather(h.reshape(_BS * _N, _D), gidx, ssc_rep)
    return out_flat.reshape(_BS, _K, _D)


# speedup vs baseline: 1.0989x; 1.0989x over previous
import functools

import numpy as np

import jax
import jax.numpy as jnp
from jax import lax
from jax.experimental import pallas as pl
from jax.experimental.pallas import tpu as pltpu
from jax.experimental.pallas import tpu_sc as plsc

_N = 50000
_K = 25000
_D = 128
_BS = 4
_NB = 2048
_NBLK = 25

_ROWS = _BS * _K
_CHUNK = 184
_WROWS = 17 * _CHUNK


def _score_body(h_ref, w_ref, b_ref, lg_ref):
    n = pl.program_id(1)
    hblk = h_ref[0]
    lg = jnp.dot(hblk.astype(jnp.bfloat16), w_ref[...].astype(jnp.bfloat16),
                 preferred_element_type=jnp.float32)[:, 0] + b_ref[0]
    s = jax.nn.sigmoid(lg)
    node = n * _NB + lax.broadcasted_iota(jnp.int32, (_NB,), 0)
    lg_ref[0] = jnp.where(node < _N, s, -1.0).reshape(_NB // _D, _D)


def _scores_l1(h, W, b):
    return pl.pallas_call(
        _score_body,
        grid=(_BS, _NBLK),
        in_specs=[
            pl.BlockSpec((1, _NB, _D), lambda bb, n: (bb, n, 0)),
            pl.BlockSpec((_D, 1), lambda bb, n: (0, 0)),
            pl.BlockSpec(memory_space=pltpu.SMEM),
        ],
        out_specs=pl.BlockSpec((1, _NB // _D, _D), lambda bb, n: (bb, n, 0)),
        out_shape=jax.ShapeDtypeStruct((_BS, 400, _D), jnp.float32),
    )(h, W, b)


def _sort_body(lg_ref, idx_ref, ssc_ref, kv_key, kv_val):
    b = pl.program_id(0)
    rl = lax.broadcasted_iota(jnp.int32, (64, 128), 0)
    cl = lax.broadcasted_iota(jnp.int32, (64, 128), 1)
    gl = rl * 128 + cl

    def ce(key, val, pk, pv, m):
        sf = (key > pk) | ((key == pk) & (val < pv))
        keep = sf == m
        return jnp.where(keep, key, pk), jnp.where(keep, val, pv)

    def stage_in(key, val, j, dirm):
        lower = (gl & j) == 0
        if j < 128:
            ax, s_lo, s_hi = 1, 128 - j, j
        else:
            d = j // 128
            ax, s_lo, s_hi = 0, 64 - d, d
        pk = jnp.where(lower, pltpu.roll(key, s_lo, ax),
                       pltpu.roll(key, s_hi, ax))
        pv = jnp.where(lower, pltpu.roll(val, s_lo, ax),
                       pltpu.roll(val, s_hi, ax))
        return ce(key, val, pk, pv, lower == dirm)

    def cross(ka, va, kb, vb, dirb):
        sf = (ka > kb) | ((ka == kb) & (va < vb))
        keep = sf == dirb
        nka, nva = jnp.where(keep, ka, kb), jnp.where(keep, va, vb)
        nkb, nvb = jnp.where(keep, kb, ka), jnp.where(keep, vb, va)
        return nka, nva, nkb, nvb

    in_sched = []
    for p in range(1, 13):
        k = 1 << p
        j = k >> 1
        while j >= 1:
            in_sched.append((j, k))
            j >>= 1
    tail = [4096 >> t for t in range(13)]

    def pass_a(cc, carry):
        c0 = 2 * cc
        k0 = lg_ref[0, pl.ds(c0 * 64, 64), :]
        v0 = gl + c0 * 8192 + b * _N
        k1 = lg_ref[0, pl.ds(c0 * 64 + 64, 64), :]
        v1 = v0 + 8192
        for (j, k) in in_sched:
            dm = (gl & k) == 0
            k0, v0 = stage_in(k0, v0, j, dm)
            k1, v1 = stage_in(k1, v1, j, dm)
        for j in tail:
            k0, v0 = stage_in(k0, v0, j, True)
            k1, v1 = stage_in(k1, v1, j, False)
        kv_key[pl.ds(c0 * 64, 64), :] = k0
        kv_val[pl.ds(c0 * 64, 64), :] = v0
        kv_key[pl.ds(c0 * 64 + 64, 64), :] = k1
        kv_val[pl.ds(c0 * 64 + 64, 64), :] = v1
        return carry

    def make_cross_pass(c0_of, dist, kshift, with_tail, to_out):
        def body(cp, carry):
            c0 = c0_of(cp)
            oa, ob = c0 * 64, (c0 + dist) * 64
            ka = kv_key[pl.ds(oa, 64), :]
            va = kv_val[pl.ds(oa, 64), :]
            kb = kv_key[pl.ds(ob, 64), :]
            vb = kv_val[pl.ds(ob, 64), :]
            dirb = True if kshift is None else ((c0 >> kshift) & 1) == 0
            ka, va, kb, vb = cross(ka, va, kb, vb, dirb)
            if with_tail:
                for j in tail:
                    ka, va = stage_in(ka, va, j, dirb)
                    kb, vb = stage_in(kb, vb, j, dirb)
            if to_out:
                ssc_ref[0, pl.ds(oa, 64), :] = ka
                idx_ref[0, pl.ds(oa, 64), :] = va
                ssc_ref[0, pl.ds(ob, 64), :] = kb
                idx_ref[0, pl.ds(ob, 64), :] = vb
            else:
                kv_key[pl.ds(oa, 64), :] = ka
                kv_val[pl.ds(oa, 64), :] = va
                kv_key[pl.ds(ob, 64), :] = kb
                kv_val[pl.ds(ob, 64), :] = vb
            return carry

        return body

    d1 = lambda cp: 2 * cp
    d2 = lambda cp: (cp & 1) + (cp >> 1) * 4
    d4 = lambda cp: cp

    lax.fori_loop(0, 4, pass_a, 0)
    lax.fori_loop(0, 4, make_cross_pass(d1, 1, 1, True, False), 0)
    lax.fori_loop(0, 4, make_cross_pass(d2, 2, 2, False, False), 0)
    lax.fori_loop(0, 4, make_cross_pass(d1, 1, 2, True, False), 0)
    lax.fori_loop(0, 4, make_cross_pass(d4, 4, None, False, False), 0)
    lax.fori_loop(0, 4, make_cross_pass(d2, 2, None, False, False), 0)
    lax.fori_loop(0, 4, make_cross_pass(d1, 1, None, True, True), 0)


def _topk_sort(scores_l1):
    f = pl.pallas_call(
        _sort_body,
        grid=(_BS,),
        in_specs=[pl.BlockSpec((1, 512, 128), lambda b: (b, 0, 0))],
        out_specs=[
            pl.BlockSpec((1, 512, 128), lambda b: (b, 0, 0)),
            pl.BlockSpec((1, 512, 128), lambda b: (b, 0, 0)),
        ],
        out_shape=[
            jax.ShapeDtypeStruct((_BS, 512, 128), jnp.int32),
            jax.ShapeDtypeStruct((_BS, 512, 128), jnp.float32),
        ],
        scratch_shapes=[
            pltpu.VMEM((512, 128), jnp.float32),
            pltpu.VMEM((512, 128), jnp.int32),
        ],
    )
    return f(scores_l1)


def _gather_body(h_hbm, gidx_hbm, ssc_hbm, out_hbm,
                 idx_a, sc0, sc1, rows0, rows1, sem0, sem1):
    wid = lax.axis_index("s") * 2 + lax.axis_index("c")
    base_w = jnp.minimum(wid * _WROWS, _ROWS - _WROWS)
    pltpu.sync_copy(gidx_hbm.at[pl.ds(base_w, _WROWS)], idx_a)
    bufs = (rows0, rows1)
    scs = (sc0, sc1)
    sems = (sem0, sem1)

    def gather(cc, par):
        rh = pltpu.async_copy(
            h_hbm.at[idx_a.at[pl.ds(cc * _CHUNK, _CHUNK)]], bufs[par],
            sems[par])
        sh = pltpu.async_copy(
            ssc_hbm.at[pl.ds(base_w + cc * _CHUNK, _CHUNK)], scs[par],
            sems[par])
        return rh, sh

    handles = [gather(0, 0), None]
    for t in range(17):
        cur = t % 2
        if t < 16:
            handles[1 - cur] = gather(t + 1, 1 - cur)
        rh, sh = handles[cur]
        rh.wait()
        sh.wait()
        buf = bufs[cur]
        sc = scs[cur]

        def scale_rows(r8, carry, _buf=buf, _sc=sc):
            for u in range(8):
                r = r8 * 8 + u
                s16 = _sc[r]
                for q in range(8):
                    _buf[r, pl.ds(q * 16, 16)] = (
                        _buf[r, pl.ds(q * 16, 16)] * s16)
            return carry

        lax.fori_loop(0, _CHUNK // 8, scale_rows, 0)
        pltpu.sync_copy(buf, out_hbm.at[pl.ds(base_w + t * _CHUNK, _CHUNK)])


def _sc_gather(h_flat, gidx, ssc):
    gk = functools.partial(
        pl.kernel,
        mesh=plsc.VectorSubcoreMesh(core_axis_name="c", subcore_axis_name="s"),
        out_type=jax.ShapeDtypeStruct((_ROWS, _D), jnp.float32),
        scratch_types=[
            pltpu.VMEM((_WROWS,), jnp.int32),
            pltpu.VMEM((_CHUNK, 16), jnp.float32),
            pltpu.VMEM((_CHUNK, 16), jnp.float32),
            pltpu.VMEM((_CHUNK, _D), jnp.float32),
            pltpu.VMEM((_CHUNK, _D), jnp.float32),
            pltpu.SemaphoreType.DMA,
            pltpu.SemaphoreType.DMA,
        ],
    )(_gather_body)
    return gk(h_flat, gidx, ssc)


def kernel(h, W, b):
    s400 = _scores_l1(h, W, b)
    scores_l1 = jnp.concatenate(
        [s400, jnp.full((_BS, 112, _D), -1.0, jnp.float32)], axis=1)
    idx_l1, ssc_l1 = _topk_sort(scores_l1)
    gidx = idx_l1.reshape(_BS, 512 * _D)[:, :_K].reshape(-1)
    ssc = ssc_l1.reshape(_BS, 512 * _D)[:, :_K].reshape(-1)
    ssc_rep = jnp.broadcast_to(ssc[:, None], (_ROWS, 16))
    out_flat = _sc_gather(h.reshape(_BS * _N, _D), gidx, ssc_rep)
    return out_flat.reshape(_BS, _K, _D)


# speedup vs baseline: 1.1000x; 1.0010x over previous
import functools

import jax
import jax.numpy as jnp
from jax import lax
from jax.experimental import pallas as pl
from jax.experimental.pallas import tpu as pltpu
from jax.experimental.pallas import tpu_sc as plsc

_N = 50000
_K = 25000
_D = 128
_BS = 4
_NB = 2048
_NBLK = 25

_ROWS = _BS * _K
_CHUNK = 184
_WROWS = 17 * _CHUNK


def _score_body(h_ref, w_ref, b_ref, lg_ref):
    n = pl.program_id(1)
    hblk = h_ref[0]
    lg = jnp.dot(hblk.astype(jnp.bfloat16), w_ref[...].astype(jnp.bfloat16),
                 preferred_element_type=jnp.float32)[:, 0] + b_ref[0]
    s = jax.nn.sigmoid(lg)
    node = n * _NB + lax.broadcasted_iota(jnp.int32, (_NB,), 0)
    lg_ref[0] = jnp.where(node < _N, s, -1.0).reshape(_NB // _D, _D)


def _scores_l1(h, W, b):
    return pl.pallas_call(
        _score_body,
        grid=(_BS, _NBLK),
        in_specs=[
            pl.BlockSpec((1, _NB, _D), lambda bb, n: (bb, n, 0)),
            pl.BlockSpec((_D, 1), lambda bb, n: (0, 0)),
            pl.BlockSpec(memory_space=pltpu.SMEM),
        ],
        out_specs=pl.BlockSpec((1, _NB // _D, _D), lambda bb, n: (bb, n, 0)),
        out_shape=jax.ShapeDtypeStruct((_BS, 400, _D), jnp.float32),
    )(h, W, b)


def _sort_body(lg_ref, idx_ref, ssc_ref, kv_key, kv_val):
    b = pl.program_id(0)
    rl = lax.broadcasted_iota(jnp.int32, (64, 128), 0)
    cl = lax.broadcasted_iota(jnp.int32, (64, 128), 1)
    gl = rl * 128 + cl

    def ce(key, val, pk, pv, m):
        sf = (key > pk) | ((key == pk) & (val < pv))
        keep = sf == m
        return jnp.where(keep, key, pk), jnp.where(keep, val, pv)

    def stage_in(key, val, j, dirm):
        lower = (gl & j) == 0
        if j < 128:
            ax, s_lo, s_hi = 1, 128 - j, j
        else:
            d = j // 128
            ax, s_lo, s_hi = 0, 64 - d, d
        pk = jnp.where(lower, pltpu.roll(key, s_lo, ax),
                       pltpu.roll(key, s_hi, ax))
        pv = jnp.where(lower, pltpu.roll(val, s_lo, ax),
                       pltpu.roll(val, s_hi, ax))
        return ce(key, val, pk, pv, lower == dirm)

    def cross(ka, va, kb, vb, dirb):
        sf = (ka > kb) | ((ka == kb) & (va < vb))
        keep = sf == dirb
        nka, nva = jnp.where(keep, ka, kb), jnp.where(keep, va, vb)
        nkb, nvb = jnp.where(keep, kb, ka), jnp.where(keep, vb, va)
        return nka, nva, nkb, nvb

    in_sched = []
    for p in range(1, 13):
        k = 1 << p
        j = k >> 1
        while j >= 1:
            in_sched.append((j, k))
            j >>= 1
    tail = [4096 >> t for t in range(13)]

    def pass_a(cc, carry):
        c0 = 2 * cc
        k0 = lg_ref[0, pl.ds(c0 * 64, 64), :]
        v0 = gl + c0 * 8192 + b * _N
        k1 = lg_ref[0, pl.ds(c0 * 64 + 64, 64), :]
        v1 = v0 + 8192
        for (j, k) in in_sched:
            dm = (gl & k) == 0
            k0, v0 = stage_in(k0, v0, j, dm)
            k1, v1 = stage_in(k1, v1, j, dm)
        for j in tail:
            k0, v0 = stage_in(k0, v0, j, True)
            k1, v1 = stage_in(k1, v1, j, False)
        kv_key[pl.ds(c0 * 64, 64), :] = k0
        kv_val[pl.ds(c0 * 64, 64), :] = v0
        kv_key[pl.ds(c0 * 64 + 64, 64), :] = k1
        kv_val[pl.ds(c0 * 64 + 64, 64), :] = v1
        return carry

    def make_cross_pass(c0_of, dist, kshift, with_tail, to_out):
        def body(cp, carry):
            c0 = c0_of(cp)
            oa, ob = c0 * 64, (c0 + dist) * 64
            ka = kv_key[pl.ds(oa, 64), :]
            va = kv_val[pl.ds(oa, 64), :]
            kb = kv_key[pl.ds(ob, 64), :]
            vb = kv_val[pl.ds(ob, 64), :]
            dirb = True if kshift is None else ((c0 >> kshift) & 1) == 0
            ka, va, kb, vb = cross(ka, va, kb, vb, dirb)
            if with_tail:
                for j in tail:
                    ka, va = stage_in(ka, va, j, dirb)
                    kb, vb = stage_in(kb, vb, j, dirb)
            if to_out:
                ssc_ref[0, pl.ds(oa, 64), :] = ka
                idx_ref[0, pl.ds(oa, 64), :] = va
                ssc_ref[0, pl.ds(ob, 64), :] = kb
                idx_ref[0, pl.ds(ob, 64), :] = vb
            else:
                kv_key[pl.ds(oa, 64), :] = ka
                kv_val[pl.ds(oa, 64), :] = va
                kv_key[pl.ds(ob, 64), :] = kb
                kv_val[pl.ds(ob, 64), :] = vb
            return carry

        return body

    d1 = lambda cp: 2 * cp
    d2 = lambda cp: (cp & 1) + (cp >> 1) * 4
    d4 = lambda cp: cp

    lax.fori_loop(0, 4, pass_a, 0)
    lax.fori_loop(0, 4, make_cross_pass(d1, 1, 1, True, False), 0)
    lax.fori_loop(0, 4, make_cross_pass(d2, 2, 2, False, False), 0)
    lax.fori_loop(0, 4, make_cross_pass(d1, 1, 2, True, False), 0)
    lax.fori_loop(0, 4, make_cross_pass(d4, 4, None, False, False), 0)
    lax.fori_loop(0, 4, make_cross_pass(d2, 2, None, False, False), 0)
    lax.fori_loop(0, 4, make_cross_pass(d1, 1, None, True, True), 0)


def _topk_sort(scores_l1):
    f = pl.pallas_call(
        _sort_body,
        grid=(_BS,),
        in_specs=[pl.BlockSpec((1, 512, 128), lambda b: (b, 0, 0))],
        out_specs=[
            pl.BlockSpec((1, 512, 128), lambda b: (b, 0, 0)),
            pl.BlockSpec((1, 512, 128), lambda b: (b, 0, 0)),
        ],
        out_shape=[
            jax.ShapeDtypeStruct((_BS, 512, 128), jnp.int32),
            jax.ShapeDtypeStruct((_BS, 512, 128), jnp.float32),
        ],
        scratch_shapes=[
            pltpu.VMEM((512, 128), jnp.float32),
            pltpu.VMEM((512, 128), jnp.int32),
        ],
    )
    return f(scores_l1)


def _gather_body(h_hbm, gidx_hbm, ssc_hbm, out_hbm,
                 idx_a, sc0, sc1, rows0, rows1, sem0, sem1):
    wid = lax.axis_index("s") * 2 + lax.axis_index("c")
    base_w = jnp.minimum(wid * _WROWS, _ROWS - _WROWS)
    pltpu.sync_copy(gidx_hbm.at[pl.ds(base_w, _WROWS)], idx_a)
    bufs = (rows0, rows1)
    scs = (sc0, sc1)
    sems = (sem0, sem1)

    def gather(cc, par):
        rh = pltpu.async_copy(
            h_hbm.at[idx_a.at[pl.ds(cc * _CHUNK, _CHUNK)]], bufs[par],
            sems[par])
        sh = pltpu.async_copy(
            ssc_hbm.at[pl.ds(base_w + cc * _CHUNK, _CHUNK)], scs[par],
            sems[par])
        return rh, sh

    handles = [gather(0, 0), None]
    for t in range(17):
        cur = t % 2
        if t < 16:
            handles[1 - cur] = gather(t + 1, 1 - cur)
        rh, sh = handles[cur]
        rh.wait()
        sh.wait()
        buf = bufs[cur]
        sc = scs[cur]

        def scale_rows(r8, carry, _buf=buf, _sc=sc):
            for u in range(8):
                r = r8 * 8 + u
                s16 = _sc[r]
                for q in range(8):
                    _buf[r, pl.ds(q * 16, 16)] = (
                        _buf[r, pl.ds(q * 16, 16)] * s16)
            return carry

        lax.fori_loop(0, _CHUNK // 8, scale_rows, 0)
        pltpu.sync_copy(buf, out_hbm.at[pl.ds(base_w + t * _CHUNK, _CHUNK)])


def _sc_gather(h_flat, gidx, ssc):
    gk = functools.partial(
        pl.kernel,
        mesh=plsc.VectorSubcoreMesh(core_axis_name="c", subcore_axis_name="s"),
        out_type=jax.ShapeDtypeStruct((_ROWS, _D), jnp.float32),
        scratch_types=[
            pltpu.VMEM((_WROWS,), jnp.int32),
            pltpu.VMEM((_CHUNK, 16), jnp.float32),
            pltpu.VMEM((_CHUNK, 16), jnp.float32),
            pltpu.VMEM((_CHUNK, _D), jnp.float32),
            pltpu.VMEM((_CHUNK, _D), jnp.float32),
            pltpu.SemaphoreType.DMA,
            pltpu.SemaphoreType.DMA,
        ],
    )(_gather_body)
    return gk(h_flat, gidx, ssc)


def kernel(h, W, b):
    s400 = _scores_l1(h, W, b)
    scores_l1 = jnp.concatenate(
        [s400, jnp.full((_BS, 112, _D), -1.0, jnp.float32)], axis=1)
    idx_l1, ssc_l1 = _topk_sort(scores_l1)
    gidx = idx_l1.reshape(_BS, 512 * _D)[:, :_K].reshape(-1)
    ssc = ssc_l1.reshape(_BS, 512 * _D)[:, :_K].reshape(-1)
    ssc_rep = jnp.broadcast_to(ssc[:, None], (_ROWS, 16))
    out_flat = _sc_gather(h.reshape(_BS * _N, _D), gidx, ssc_rep)
    return out_flat.reshape(_BS, _K, _D)
